# Initial kernel scaffold; baseline (speedup 1.0000x reference)
import jax
import jax.numpy as jnp
from jax.experimental import pallas as pl


def kernel(x, pred_class, W1, b1, g1, bt1, Wr1, br1, gr1, btr1, Wr2, br2, gr2, btr2, Wf, bf):
    raise NotImplementedError("write your pallas kernel here")


# speedup vs baseline: 8886.7427x; 8886.7427x over previous
import functools

import numpy as np
import jax
import jax.numpy as jnp
from jax.experimental import pallas as pl
from jax.experimental.pallas import tpu as pltpu

_NPOSE = 216
_ODIM = 229
_EPS = 1e-5
_NITER = 4
_PREC = jax.lax.Precision.DEFAULT
_PREC_PERM = jax.lax.Precision.HIGHEST


def _interleave_mat() -> np.ndarray:
    P = np.zeros((_NPOSE, _NPOSE), dtype=np.float32)
    for p in range(9):
        for j in range(24):
            P[p * 24 + j, j * 9 + p] = 1.0
    return P


def _bn_apply(y, st0, st1, g, bt, n):
    mu = jnp.sum(st0, axis=0, keepdims=True) / n
    ey2 = jnp.sum(st1, axis=0, keepdims=True) / n
    var = ey2 - mu * mu
    return (y - mu) / jnp.sqrt(var + _EPS) * g + bt


def _polar_t(poseT):
    X = [poseT[24 * p:24 * (p + 1), :] for p in range(9)]
    s = None
    for _ in range(_NITER):
        c0 = X[4] * X[8] - X[5] * X[7]
        c1 = X[5] * X[6] - X[3] * X[8]
        c2 = X[3] * X[7] - X[4] * X[6]
        c3 = X[2] * X[7] - X[1] * X[8]
        c4 = X[0] * X[8] - X[2] * X[6]
        c5 = X[1] * X[6] - X[0] * X[7]
        c6 = X[1] * X[5] - X[2] * X[4]
        c7 = X[2] * X[3] - X[0] * X[5]
        c8 = X[0] * X[4] - X[1] * X[3]
        dd = X[0] * c0 + X[1] * c1 + X[2] * c2
        ad = jnp.maximum(jnp.abs(dd), 1e-30)
        if s is None:
            s = jnp.where(dd < 0, -1.0, 1.0)
        g = jnp.exp(jnp.log(ad) * (-1.0 / 3.0))
        inv = 0.5 / (g * jnp.where(dd < 0, -ad, ad))
        gh = 0.5 * g
        C = (c0, c1, c2, c3, c4, c5, c6, c7, c8)
        X = [gh * X[p] + C[p] * inv for p in range(9)]
    return jnp.concatenate([x * s for x in X], axis=0)


def _fused_body(n_rows, tile, x_ref, w1_ref, b1_ref, g1_ref, bt1_ref,
                wr1_ref, br1_ref, gr1_ref, btr1_ref,
                wr2_ref, br2_ref, gr2_ref, btr2_ref,
                wf_ref, bf_ref, p_ref, rot_ref, bet_ref, cam_ref,
                h_scr, y_scr, st_scr):
    p = pl.program_id(0)
    t = pl.program_id(1)
    rows = pl.ds(t * tile, tile)

    @pl.when(p == 0)
    def _():
        y = jnp.dot(x_ref[...], w1_ref[...],
                    preferred_element_type=jnp.float32,
                    precision=_PREC) + b1_ref[...]
        y_scr[rows, :] = y
        st_scr[pl.ds(t, 1), :] = jnp.sum(y, axis=0, keepdims=True)
        st_scr[pl.ds(t + 8, 1), :] = jnp.sum(y * y, axis=0, keepdims=True)

    @pl.when(p == 1)
    def _():
        h = jnp.maximum(
            _bn_apply(y_scr[rows, :], st_scr[0:8, :], st_scr[8:16, :],
                      g1_ref[...], bt1_ref[...], n_rows), 0.0)
        h_scr[rows, :] = h
        y = jnp.dot(h, wr1_ref[...], preferred_element_type=jnp.float32,
                    precision=_PREC) + br1_ref[...]
        y_scr[rows, :] = y
        st_scr[pl.ds(t + 16, 1), :] = jnp.sum(y, axis=0, keepdims=True)
        st_scr[pl.ds(t + 24, 1), :] = jnp.sum(y * y, axis=0, keepdims=True)

    @pl.when(p == 2)
    def _():
        r = jnp.maximum(
            _bn_apply(y_scr[rows, :], st_scr[16:24, :], st_scr[24:32, :],
                      gr1_ref[...], btr1_ref[...], n_rows), 0.0)
        y = jnp.dot(r, wr2_ref[...], preferred_element_type=jnp.float32,
                    precision=_PREC) + br2_ref[...]
        y_scr[rows, :] = y
        st_scr[pl.ds(t + 32, 1), :] = jnp.sum(y, axis=0, keepdims=True)
        st_scr[pl.ds(t + 40, 1), :] = jnp.sum(y * y, axis=0, keepdims=True)

    @pl.when(p == 3)
    def _():
        r = _bn_apply(y_scr[rows, :], st_scr[32:40, :], st_scr[40:48, :],
                      gr2_ref[...], btr2_ref[...], n_rows)
        h2 = jnp.maximum(h_scr[rows, :] + r, 0.0)
        h2a = jnp.concatenate(
            [h2, jnp.ones((h2.shape[0], 1), jnp.float32)], axis=1)
        wfa = jnp.concatenate([wf_ref[...], bf_ref[...]], axis=0)
        s_jm = jax.lax.dot_general(
            wfa[:, :_NPOSE], h2a, (((0,), (1,)), ((), ())),
            preferred_element_type=jnp.float32, precision=_PREC)
        poseT = jnp.dot(p_ref[...], s_jm, preferred_element_type=jnp.float32,
                        precision=_PREC_PERM)
        planesT = _polar_t(poseT)
        rot_ref[...] = jax.lax.dot_general(
            planesT, p_ref[...], (((0,), (0,)), ((), ())),
            preferred_element_type=jnp.float32, precision=_PREC_PERM)
        bc = jnp.dot(h2a, wfa[:, _NPOSE:], preferred_element_type=jnp.float32,
                     precision=_PREC)
        bet_ref[...] = bc[:, :10]
        cam_ref[...] = bc[:, 10:]


def kernel(x, pred_class, W1, b1, g1, bt1, Wr1, br1, gr1, btr1,
           Wr2, br2, gr2, btr2, Wf, bf):
    stage, bs, nq, ch = x.shape
    n = stage * bs * nq
    tile = 1800
    nt = n // tile

    xf = x.reshape(n, ch)
    row = lambda v: v.reshape(1, -1)
    pmat = jnp.asarray(_interleave_mat())

    x_spec = pl.BlockSpec((tile, ch),
                          lambda p, t: (jnp.where(p == 0, t, 0), 0))
    w_spec = pl.BlockSpec((ch, ch), lambda p, t: (0, 0))
    v_spec = pl.BlockSpec((1, ch), lambda p, t: (0, 0))

    def out_spec(width):
        return pl.BlockSpec((tile, width),
                            lambda p, t: (jnp.where(p == 3, t, 0), 0))

    rot, betas, camera = pl.pallas_call(
        functools.partial(_fused_body, float(n), tile),
        grid=(4, nt),
        in_specs=[x_spec, w_spec, v_spec, v_spec, v_spec,
                  w_spec, v_spec, v_spec, v_spec,
                  w_spec, v_spec, v_spec, v_spec,
                  pl.BlockSpec((ch, _ODIM), lambda p, t: (0, 0)),
                  pl.BlockSpec((1, _ODIM), lambda p, t: (0, 0)),
                  pl.BlockSpec((_NPOSE, _NPOSE), lambda p, t: (0, 0))],
        out_specs=[out_spec(_NPOSE), out_spec(10), out_spec(3)],
        out_shape=[jax.ShapeDtypeStruct((n, _NPOSE), jnp.float32),
                   jax.ShapeDtypeStruct((n, 10), jnp.float32),
                   jax.ShapeDtypeStruct((n, 3), jnp.float32)],
        scratch_shapes=[pltpu.VMEM((n, ch), jnp.float32),
                        pltpu.VMEM((n, ch), jnp.float32),
                        pltpu.VMEM((48, ch), jnp.float32)],
        compiler_params=pltpu.CompilerParams(
            dimension_semantics=("arbitrary", "arbitrary")),
    )(xf, W1, row(b1), row(g1), row(bt1),
      Wr1, row(br1), row(gr1), row(btr1),
      Wr2, row(br2), row(gr2), row(btr2),
      Wf, row(bf), pmat)

    rotmat = rot.reshape(stage, bs, nq, 24, 3, 3)
    betas = betas.reshape(stage, bs, nq, 10)
    camera = camera.reshape(stage, bs, nq, 3)
    return (rotmat, betas, camera)
